# baseline (speedup 1.0000x reference)
import functools

import jax
import jax.numpy as jnp
from jax import lax
from jax.experimental import pallas as pl
from jax.experimental.pallas import tpu as pltpu
from jax.experimental.pallas import tpu_sc as plsc

B = 8
NV = 4096
NG = 256
NC = B * NG
NW = 32
NH = NV // 2
RPW = NH // NW
CH = 8
NCHUNK = RPW // CH
EPS = 1e-6


_MESH = plsc.VectorSubcoreMesh(core_axis_name="c", subcore_axis_name="s")


def _make_scgather(h):
    @functools.partial(
        pl.kernel,
        mesh=_MESH,
        compiler_params=pltpu.CompilerParams(needs_layout_passes=False),
        out_type=jax.ShapeDtypeStruct((NH, NC), jnp.float32),
        scratch_types=[
            pltpu.VMEM((CH, NV), jnp.float32),
            pltpu.VMEM((CH, NV), jnp.float32),
            pltpu.VMEM((CH, NC), jnp.float32),
            pltpu.VMEM((CH, NC), jnp.float32),
            pltpu.VMEM((NC,), jnp.int32),
            pltpu.SemaphoreType.DMA,
            pltpu.SemaphoreType.DMA,
            pltpu.SemaphoreType.DMA,
            pltpu.SemaphoreType.DMA,
        ],
    )
    def _scgather(dis_hbm, gt_hbm, g_out,
                  rowbuf0, rowbuf1, stage0, stage1, idxbuf,
                  insem0, insem1, outsem0, outsem1):
        c = lax.axis_index("c")
        s = lax.axis_index("s")
        wid = c * 16 + s
        r0 = h * NH + wid * RPW
        o0 = wid * RPW

        pltpu.sync_copy(gt_hbm, idxbuf)

        rowbufs = (rowbuf0, rowbuf1)
        stages = (stage0, stage1)
        insems = (insem0, insem1)
        outsems = (outsem0, outsem1)
        rsplats = [jnp.full((16,), r, dtype=jnp.int32) for r in range(CH)]

        def start_in(ci, buf, sem):
            pltpu.async_copy(dis_hbm.at[pl.ds(r0 + ci * CH, CH)], buf, sem)

        start_in(0, rowbufs[0], insems[0])

        def chunk_pair(half, _):
            for p in range(2):
                ci = half * 2 + p
                nxt = (p + 1) % 2

                @pl.when(ci + 1 < NCHUNK)
                def _start(ci=ci, nxt=nxt):
                    start_in(ci + 1, rowbufs[nxt], insems[nxt])

                pltpu.make_async_copy(
                    dis_hbm.at[pl.ds(r0 + ci * CH, CH)], rowbufs[p], insems[p]
                ).wait()
                @pl.when(ci >= 2)
                def _drain(ci=ci, p=p):
                    pltpu.make_async_copy(
                        stages[p], g_out.at[pl.ds(o0 + (ci - 2) * CH, CH)],
                        outsems[p],
                    ).wait()

                def k_body(k4, __, p=p):
                    for u in range(4):
                        k = k4 * 4 + u
                        cvec = idxbuf[pl.ds(k * 16, 16)]
                        for r in range(CH):
                            g = plsc.load_gather(
                                rowbufs[p], [rsplats[r], cvec])
                            stages[p][r, pl.ds(k * 16, 16)] = g
                    return __

                lax.fori_loop(0, NC // 64, k_body, jnp.int32(0))
                pltpu.async_copy(
                    stages[p], g_out.at[pl.ds(o0 + ci * CH, CH)], outsems[p])
            return jnp.int32(0)

        lax.fori_loop(0, NCHUNK // 2, chunk_pair, jnp.int32(0))

        for p in range(2):
            ci = NCHUNK - 2 + p
            pltpu.make_async_copy(
                stages[p], g_out.at[pl.ds(o0 + ci * CH, CH)], outsems[p]
            ).wait()

    return _scgather


_scgather0 = _make_scgather(0)
_scgather1 = _make_scgather(1)


def _prep_body(pmap_ref, dis_ref, pmt_ref, qt_ref, nest_ref, m_ref):
    i = pl.program_id(0)
    nsteps = pl.num_programs(0)
    blockmax = jnp.max(dis_ref[...])
    prev = jnp.where(i == 0, -jnp.inf, m_ref[0, 0])
    cur = jnp.maximum(prev, blockmax)
    m_ref[0, 0] = cur

    @pl.when(i == nsteps - 1)
    def _():
        fp = jnp.sqrt(jnp.sum(pmap_ref[...] * pmap_ref[...], axis=2))
        pmax = jnp.max(fp, axis=1, keepdims=True)
        pm = fp / pmax
        pmt = pm.T
        pmt_ref[...] = pmt
        qt_ref[...] = (1.0 - pmt) * cur + EPS
        nest_ref[...] = jnp.sum(pm, axis=1)[None, :]


def _prep(prob_map, dis_matrix):
    blk = 512
    grid = NV // blk
    return pl.pallas_call(
        _prep_body,
        grid=(grid,),
        compiler_params=pltpu.CompilerParams(
            vmem_limit_bytes=100 * 1024 * 1024),
        in_specs=[
            pl.BlockSpec((B, NV, 4), lambda i: (0, 0, 0)),
            pl.BlockSpec((blk, NV), lambda i: (i, 0)),
        ],
        out_specs=[
            pl.BlockSpec((NV, B), lambda i: (0, 0)),
            pl.BlockSpec((NV, B), lambda i: (0, 0)),
            pl.BlockSpec((1, B), lambda i: (0, 0)),
            pl.BlockSpec((1, 1), lambda i: (0, 0), memory_space=pltpu.SMEM),
        ],
        out_shape=[
            jax.ShapeDtypeStruct((NV, B), jnp.float32),
            jax.ShapeDtypeStruct((NV, B), jnp.float32),
            jax.ShapeDtypeStruct((1, B), jnp.float32),
            jax.ShapeDtypeStruct((1, 1), jnp.float32),
        ],
    )(prob_map, dis_matrix)


_FBLK = 512
_FSTEPS = NH // _FBLK


def _acc_block(g_ref, pmt_ref, qt_ref):
    pmt = pmt_ref[...]
    qt = qt_ref[...]
    pme = jnp.concatenate(
        [jnp.broadcast_to(pmt[:, b:b + 1], (_FBLK, NG)) for b in range(B)],
        axis=1)
    qe = jnp.concatenate(
        [jnp.broadcast_to(qt[:, b:b + 1], (_FBLK, NG)) for b in range(B)],
        axis=1)
    g = g_ref[...]
    rec = 1.0 / (g * pme + qe)
    cs_blk = jnp.sum(rec, axis=0, keepdims=True)
    gmin = jnp.min(g.reshape(_FBLK, B, NG), axis=2)
    t1_blk = jnp.sum(pmt * gmin, axis=0, keepdims=True)
    return cs_blk, t1_blk


def _final0_body(g_ref, pmt_ref, qt_ref, cs_out, t1_out):
    i = pl.program_id(0)
    cs_blk, t1_blk = _acc_block(g_ref, pmt_ref, qt_ref)
    cs_prev = jnp.where(i == 0, 0.0, cs_out[...])
    cs_out[...] = cs_prev + cs_blk
    t1_prev = jnp.where(i == 0, 0.0, t1_out[...])
    t1_out[...] = t1_prev + t1_blk


def _final0(g, pmt, qt):
    return pl.pallas_call(
        _final0_body,
        grid=(_FSTEPS,),
        in_specs=[
            pl.BlockSpec((_FBLK, NC), lambda i: (i, 0)),
            pl.BlockSpec((_FBLK, B), lambda i: (i, 0)),
            pl.BlockSpec((_FBLK, B), lambda i: (i, 0)),
        ],
        out_specs=[
            pl.BlockSpec((1, NC), lambda i: (0, 0)),
            pl.BlockSpec((1, B), lambda i: (0, 0)),
        ],
        out_shape=[
            jax.ShapeDtypeStruct((1, NC), jnp.float32),
            jax.ShapeDtypeStruct((1, B), jnp.float32),
        ],
    )(g, pmt, qt)


def _final1_body(g_ref, pmt_ref, qt_ref, cs_in, t1_in, nest_ref, out_ref,
                 cs_scr, t1_scr):
    i = pl.program_id(0)
    cs_blk, t1_blk = _acc_block(g_ref, pmt_ref, qt_ref)
    cs_prev = jnp.where(i == 0, cs_in[...], cs_scr[...])
    cs_scr[...] = cs_prev + cs_blk
    t1_prev = jnp.where(i == 0, t1_in[...], t1_scr[...])
    t1_scr[...] = t1_prev + t1_blk

    @pl.when(i == _FSTEPS - 1)
    def _():
        term2 = jnp.sum(float(NV) / cs_scr[...]) * (1.0 / (NG * B))
        term1 = jnp.sum(t1_scr[...] / (nest_ref[...] + EPS)) * (1.0 / B)
        out_ref[0, 0] = term1 + term2


def _final1(g, pmt, qt, cs0, t10, nest):
    return pl.pallas_call(
        _final1_body,
        grid=(_FSTEPS,),
        in_specs=[
            pl.BlockSpec((_FBLK, NC), lambda i: (i, 0)),
            pl.BlockSpec((_FBLK, B), lambda i: (i + _FSTEPS, 0)),
            pl.BlockSpec((_FBLK, B), lambda i: (i + _FSTEPS, 0)),
            pl.BlockSpec((1, NC), lambda i: (0, 0)),
            pl.BlockSpec((1, B), lambda i: (0, 0)),
            pl.BlockSpec((1, B), lambda i: (0, 0)),
        ],
        out_specs=pl.BlockSpec((1, 1), lambda i: (0, 0),
                               memory_space=pltpu.SMEM),
        out_shape=jax.ShapeDtypeStruct((1, 1), jnp.float32),
        scratch_shapes=[
            pltpu.VMEM((1, NC), jnp.float32),
            pltpu.VMEM((1, B), jnp.float32),
        ],
    )(g, pmt, qt, cs0, t10, nest)


def kernel(prob_map, gt, dis_matrix):
    gt_flat = gt.reshape(-1)
    g0 = _scgather0(dis_matrix, gt_flat)
    g1 = _scgather1(dis_matrix, gt_flat)
    pmt, qt, nest, _m = _prep(prob_map, dis_matrix)
    cs0, t10 = _final0(g0, pmt[:NH], qt[:NH])
    res = _final1(g1, pmt, qt, cs0, t10, nest)
    return res[0, 0]


# speedup vs baseline: 1.1572x; 1.1572x over previous
import functools

import jax
import jax.numpy as jnp
from jax import lax
from jax.experimental import pallas as pl
from jax.experimental.pallas import tpu as pltpu
from jax.experimental.pallas import tpu_sc as plsc

B = 8
NV = 4096
NG = 256
NC = B * NG
NW = 32
NH = NV // 2
RPW = NH // NW
CH = 8
NCHUNK = RPW // CH
EPS = 1e-6


_MESH = plsc.VectorSubcoreMesh(core_axis_name="c", subcore_axis_name="s")


def _make_scgather(h):
    @functools.partial(
        pl.kernel,
        mesh=_MESH,
        compiler_params=pltpu.CompilerParams(needs_layout_passes=False),
        out_type=jax.ShapeDtypeStruct((NH, NC // 2), jnp.int32),
        scratch_types=[
            pltpu.VMEM((CH, NV), jnp.float32),
            pltpu.VMEM((CH, NV), jnp.float32),
            pltpu.VMEM((CH, NC // 2), jnp.int32),
            pltpu.VMEM((CH, NC // 2), jnp.int32),
            pltpu.VMEM((NC,), jnp.int32),
            pltpu.SemaphoreType.DMA,
            pltpu.SemaphoreType.DMA,
            pltpu.SemaphoreType.DMA,
            pltpu.SemaphoreType.DMA,
        ],
    )
    def _scgather(dis_hbm, gt_hbm, g_out,
                  rowbuf0, rowbuf1, stage0, stage1, idxbuf,
                  insem0, insem1, outsem0, outsem1):
        c = lax.axis_index("c")
        s = lax.axis_index("s")
        wid = c * 16 + s
        r0 = h * NH + wid * RPW
        o0 = wid * RPW

        pltpu.sync_copy(gt_hbm, idxbuf)

        rowbufs = (rowbuf0, rowbuf1)
        stages = (stage0, stage1)
        insems = (insem0, insem1)
        outsems = (outsem0, outsem1)
        rsplats = [jnp.full((16,), r, dtype=jnp.int32) for r in range(CH)]

        def start_in(ci, buf, sem):
            pltpu.async_copy(dis_hbm.at[pl.ds(r0 + ci * CH, CH)], buf, sem)

        start_in(0, rowbufs[0], insems[0])

        def chunk_pair(half, _):
            for p in range(2):
                ci = half * 2 + p
                nxt = (p + 1) % 2

                @pl.when(ci + 1 < NCHUNK)
                def _start(ci=ci, nxt=nxt):
                    start_in(ci + 1, rowbufs[nxt], insems[nxt])

                pltpu.make_async_copy(
                    dis_hbm.at[pl.ds(r0 + ci * CH, CH)], rowbufs[p], insems[p]
                ).wait()
                @pl.when(ci >= 2)
                def _drain(ci=ci, p=p):
                    pltpu.make_async_copy(
                        stages[p], g_out.at[pl.ds(o0 + (ci - 2) * CH, CH)],
                        outsems[p],
                    ).wait()

                def k_body(k2, __, p=p):
                    for u in range(2):
                        k = (k2 * 2 + u) * 2
                        cvec0 = idxbuf[pl.ds(k * 16, 16)]
                        cvec1 = idxbuf[pl.ds((k + 1) * 16, 16)]
                        for r in range(CH):
                            ga = plsc.load_gather(
                                rowbufs[p], [rsplats[r], cvec0])
                            gb = plsc.load_gather(
                                rowbufs[p], [rsplats[r], cvec1])
                            packed = plsc.bitcast(
                                plsc.pack(
                                    ga, gb,
                                    format=plsc.PackFormat.INTERLEAVED),
                                jnp.int32)
                            stages[p][r, pl.ds(k * 8, 16)] = packed
                    return __

                lax.fori_loop(0, NC // 64, k_body, jnp.int32(0))
                pltpu.async_copy(
                    stages[p], g_out.at[pl.ds(o0 + ci * CH, CH)], outsems[p])
            return jnp.int32(0)

        lax.fori_loop(0, NCHUNK // 2, chunk_pair, jnp.int32(0))

        for p in range(2):
            ci = NCHUNK - 2 + p
            pltpu.make_async_copy(
                stages[p], g_out.at[pl.ds(o0 + ci * CH, CH)], outsems[p]
            ).wait()

    return _scgather


_scgather0 = _make_scgather(0)
_scgather1 = _make_scgather(1)


def _prep_body(pmap_ref, dis_ref, pmt_ref, qt_ref, nest_ref, m_ref):
    i = pl.program_id(0)
    nsteps = pl.num_programs(0)
    blockmax = jnp.max(dis_ref[...])
    prev = jnp.where(i == 0, -jnp.inf, m_ref[0, 0])
    cur = jnp.maximum(prev, blockmax)
    m_ref[0, 0] = cur

    @pl.when(i == nsteps - 1)
    def _():
        fp = jnp.sqrt(jnp.sum(pmap_ref[...] * pmap_ref[...], axis=2))
        pmax = jnp.max(fp, axis=1, keepdims=True)
        pm = fp / pmax
        pmt = pm.T
        pmt_ref[...] = pmt
        qt_ref[...] = (1.0 - pmt) * cur + EPS
        nest_ref[...] = jnp.sum(pm, axis=1)[None, :]


def _prep(prob_map, dis_matrix):
    blk = 512
    grid = NV // blk
    return pl.pallas_call(
        _prep_body,
        grid=(grid,),
        compiler_params=pltpu.CompilerParams(
            vmem_limit_bytes=100 * 1024 * 1024),
        in_specs=[
            pl.BlockSpec((B, NV, 4), lambda i: (0, 0, 0)),
            pl.BlockSpec((blk, NV), lambda i: (i, 0)),
        ],
        out_specs=[
            pl.BlockSpec((NV, B), lambda i: (0, 0)),
            pl.BlockSpec((NV, B), lambda i: (0, 0)),
            pl.BlockSpec((1, B), lambda i: (0, 0)),
            pl.BlockSpec((1, 1), lambda i: (0, 0), memory_space=pltpu.SMEM),
        ],
        out_shape=[
            jax.ShapeDtypeStruct((NV, B), jnp.float32),
            jax.ShapeDtypeStruct((NV, B), jnp.float32),
            jax.ShapeDtypeStruct((1, B), jnp.float32),
            jax.ShapeDtypeStruct((1, 1), jnp.float32),
        ],
    )(prob_map, dis_matrix)


_FBLK = 512
_FSTEPS = NH // _FBLK


def _acc_block(g_ref, pmt_ref, qt_ref):
    hg = NG // 2
    pmt = pmt_ref[...]
    qt = qt_ref[...]
    pme = jnp.concatenate(
        [jnp.broadcast_to(pmt[:, b:b + 1], (_FBLK, hg)) for b in range(B)],
        axis=1)
    qe = jnp.concatenate(
        [jnp.broadcast_to(qt[:, b:b + 1], (_FBLK, hg)) for b in range(B)],
        axis=1)
    g32 = g_ref[...]
    ga = lax.bitcast_convert_type(
        lax.shift_left(g32, 16), jnp.float32)
    gb = lax.bitcast_convert_type(
        lax.bitwise_and(g32, jnp.int32(-65536)), jnp.float32)
    reca = 1.0 / (ga * pme + qe)
    recb = 1.0 / (gb * pme + qe)
    cs_blk = jnp.concatenate(
        [jnp.sum(reca, axis=0, keepdims=True),
         jnp.sum(recb, axis=0, keepdims=True)], axis=1)
    gmin = jnp.minimum(
        jnp.min(ga.reshape(_FBLK, B, hg), axis=2),
        jnp.min(gb.reshape(_FBLK, B, hg), axis=2))
    t1_blk = jnp.sum(pmt * gmin, axis=0, keepdims=True)
    return cs_blk, t1_blk


def _final0_body(g_ref, pmt_ref, qt_ref, cs_out, t1_out):
    i = pl.program_id(0)
    cs_blk, t1_blk = _acc_block(g_ref, pmt_ref, qt_ref)
    cs_prev = jnp.where(i == 0, 0.0, cs_out[...])
    cs_out[...] = cs_prev + cs_blk
    t1_prev = jnp.where(i == 0, 0.0, t1_out[...])
    t1_out[...] = t1_prev + t1_blk


def _final0(g, pmt, qt):
    return pl.pallas_call(
        _final0_body,
        grid=(_FSTEPS,),
        in_specs=[
            pl.BlockSpec((_FBLK, NC // 2), lambda i: (i, 0)),
            pl.BlockSpec((_FBLK, B), lambda i: (i, 0)),
            pl.BlockSpec((_FBLK, B), lambda i: (i, 0)),
        ],
        out_specs=[
            pl.BlockSpec((1, NC), lambda i: (0, 0)),
            pl.BlockSpec((1, B), lambda i: (0, 0)),
        ],
        out_shape=[
            jax.ShapeDtypeStruct((1, NC), jnp.float32),
            jax.ShapeDtypeStruct((1, B), jnp.float32),
        ],
    )(g, pmt, qt)


def _final1_body(g_ref, pmt_ref, qt_ref, cs_in, t1_in, nest_ref, out_ref,
                 cs_scr, t1_scr):
    i = pl.program_id(0)
    cs_blk, t1_blk = _acc_block(g_ref, pmt_ref, qt_ref)
    cs_prev = jnp.where(i == 0, cs_in[...], cs_scr[...])
    cs_scr[...] = cs_prev + cs_blk
    t1_prev = jnp.where(i == 0, t1_in[...], t1_scr[...])
    t1_scr[...] = t1_prev + t1_blk

    @pl.when(i == _FSTEPS - 1)
    def _():
        term2 = jnp.sum(float(NV) / cs_scr[...]) * (1.0 / (NG * B))
        term1 = jnp.sum(t1_scr[...] / (nest_ref[...] + EPS)) * (1.0 / B)
        out_ref[0, 0] = term1 + term2


def _final1(g, pmt, qt, cs0, t10, nest):
    return pl.pallas_call(
        _final1_body,
        grid=(_FSTEPS,),
        in_specs=[
            pl.BlockSpec((_FBLK, NC // 2), lambda i: (i, 0)),
            pl.BlockSpec((_FBLK, B), lambda i: (i + _FSTEPS, 0)),
            pl.BlockSpec((_FBLK, B), lambda i: (i + _FSTEPS, 0)),
            pl.BlockSpec((1, NC), lambda i: (0, 0)),
            pl.BlockSpec((1, B), lambda i: (0, 0)),
            pl.BlockSpec((1, B), lambda i: (0, 0)),
        ],
        out_specs=pl.BlockSpec((1, 1), lambda i: (0, 0),
                               memory_space=pltpu.SMEM),
        out_shape=jax.ShapeDtypeStruct((1, 1), jnp.float32),
        scratch_shapes=[
            pltpu.VMEM((1, NC), jnp.float32),
            pltpu.VMEM((1, B), jnp.float32),
        ],
    )(g, pmt, qt, cs0, t10, nest)


def kernel(prob_map, gt, dis_matrix):
    gt_flat = gt.reshape(-1)
    g0 = _scgather0(dis_matrix, gt_flat)
    g1 = _scgather1(dis_matrix, gt_flat)
    pmt, qt, nest, _m = _prep(prob_map, dis_matrix)
    cs0, t10 = _final0(g0, pmt[:NH], qt[:NH])
    res = _final1(g1, pmt, qt, cs0, t10, nest)
    return res[0, 0]
